# Initial kernel scaffold; baseline (speedup 1.0000x reference)
import functools

import jax
import jax.numpy as jnp
from jax import lax
from jax.experimental import pallas as pl
from jax.experimental.pallas import tpu as pltpu
from jax.experimental.pallas import tpu_sc as plsc

N = 10000
E = 320000
F_IN = 128
H = 64
G = 64

NC, NS = 2, 16
NW = NC * NS
EPW = E // NW
CH = 80
NCHUNK = EPW // CH
RPT = N // NS
DW = 16

BK = 240
NPAD = N + BK

_SC_MESH = plsc.VectorSubcoreMesh(core_axis_name="c", subcore_axis_name="s")


def _sc_agg(hs, srcr, dstr, zrows):
    @functools.partial(
        pl.kernel,
        out_type=jax.ShapeDtypeStruct((NC, N, H), jnp.float32),
        mesh=_SC_MESH,
        scratch_types=[
            pltpu.VMEM((NCHUNK, CH), jnp.int32),
            pltpu.VMEM((NCHUNK, CH), jnp.int32),
            pltpu.VMEM((CH, H), jnp.float32),
            pltpu.VMEM_SHARED((N, H), jnp.float32),
        ],
    )
    def k(hs_hbm, src_hbm, dst_hbm, z_hbm, out_hbm, sslab, dslab, rows, acc):
        cid = lax.axis_index("c")
        sid = lax.axis_index("s")
        wid = sid * NC + cid
        r0 = sid * RPT

        pltpu.sync_copy(z_hbm, acc.at[pl.ds(r0, RPT)])
        pltpu.sync_copy(src_hbm.at[wid], sslab)
        pltpu.sync_copy(dst_hbm.at[wid], dslab)
        plsc.subcore_barrier()

        @pl.loop(0, NCHUNK)
        def _(j):
            pltpu.sync_copy(hs_hbm.at[sslab.at[j]], rows)
            pltpu.sync_copy(rows, acc.at[dslab.at[j]], add=True)

        plsc.subcore_barrier()
        pltpu.sync_copy(acc.at[pl.ds(r0, RPT)], out_hbm.at[cid, pl.ds(r0, RPT)])

    return k(hs, srcr, dstr, zrows)


def _sc_deg(dstr, ones, zrows):
    @functools.partial(
        pl.kernel,
        out_type=jax.ShapeDtypeStruct((NC, N, DW), jnp.float32),
        mesh=_SC_MESH,
        scratch_types=[
            pltpu.VMEM((NCHUNK, CH), jnp.int32),
            pltpu.VMEM((CH, DW), jnp.float32),
            pltpu.VMEM_SHARED((N, DW), jnp.float32),
        ],
    )
    def k(dst_hbm, ones_hbm, z_hbm, out_hbm, dslab, ones_v, acc):
        cid = lax.axis_index("c")
        sid = lax.axis_index("s")
        wid = sid * NC + cid
        r0 = sid * RPT

        pltpu.sync_copy(z_hbm, acc.at[pl.ds(r0, RPT)])
        pltpu.sync_copy(ones_hbm, ones_v)
        pltpu.sync_copy(dst_hbm.at[wid], dslab)
        plsc.subcore_barrier()

        @pl.loop(0, NCHUNK)
        def _(j):
            pltpu.sync_copy(ones_v, acc.at[dslab.at[j]], add=True)

        plsc.subcore_barrier()
        pltpu.sync_copy(acc.at[pl.ds(r0, RPT)], out_hbm.at[cid, pl.ds(r0, RPT)])

    return k(dstr, ones, zrows)


def _tc_mm(x, w):
    def body(x_ref, w_ref, o_ref):
        o_ref[...] = jnp.dot(x_ref[...], w_ref[...],
                             preferred_element_type=jnp.float32)

    return pl.pallas_call(
        body, out_shape=jax.ShapeDtypeStruct((N, H), jnp.float32))(x, w)


def _tc_dinv_scale(z1, degp):
    def body(z_ref, d_ref, dinv_ref, hs_ref):
        deg = 1.0 + d_ref[0, :, 0:1] + d_ref[1, :, 0:1]
        dinv = lax.rsqrt(deg)
        dinv_ref[...] = dinv
        hs_ref[...] = z_ref[...] * dinv

    return pl.pallas_call(
        body,
        out_shape=(jax.ShapeDtypeStruct((N, 1), jnp.float32),
                   jax.ShapeDtypeStruct((N, H), jnp.float32)))(z1, degp)


def _tc_layer(ap, hs, dinv, b, g, be, wn):
    def body(p_ref, hs_ref, dinv_ref, b_ref, g_ref, be_ref, w_ref, o_ref):
        dv = dinv_ref[...]
        agg = dv * (p_ref[0] + p_ref[1] + hs_ref[...]) + b_ref[...]
        m = jnp.mean(agg, axis=0, keepdims=True)
        c = agg - m
        v = jnp.mean(c * c, axis=0, keepdims=True)
        hn = jnp.maximum(c * lax.rsqrt(v + 1e-5) * g_ref[...] + be_ref[...],
                         0.0)
        z = jnp.dot(hn, w_ref[...], preferred_element_type=jnp.float32)
        o_ref[...] = z * dv

    return pl.pallas_call(
        body, out_shape=jax.ShapeDtypeStruct((N, H), jnp.float32))(
            ap, hs, dinv, b, g, be, wn)


def _tc_last_layer(ap, hs, dinv, b, g, be, batch2):
    def body(p_ref, hs_ref, dinv_ref, b_ref, g_ref, be_ref, bt_ref,
             h3_ref, co_ref):
        agg = (dinv_ref[...] * (p_ref[0] + p_ref[1] + hs_ref[...])
               + b_ref[...])
        m = jnp.mean(agg, axis=0, keepdims=True)
        c = agg - m
        v = jnp.mean(c * c, axis=0, keepdims=True)
        h3 = jnp.maximum(c * lax.rsqrt(v + 1e-5) * g_ref[...] + be_ref[...],
                         0.0)
        h3_ref[0:N, :] = h3
        h3_ref[N:NPAD, :] = jnp.zeros((NPAD - N, H), jnp.float32)
        bt = bt_ref[...]
        gi = lax.broadcasted_iota(jnp.int32, (N, G), 1)
        co_ref[0:1, :] = jnp.sum((bt == gi).astype(jnp.int32), axis=0,
                                 keepdims=True)
        co_ref[1:2, :] = jnp.sum((bt < gi).astype(jnp.int32), axis=0,
                                 keepdims=True)

    return pl.pallas_call(
        body,
        out_shape=(jax.ShapeDtypeStruct((NPAD, H), jnp.float32),
                   jax.ShapeDtypeStruct((2, G), jnp.int32)))(
            ap, hs, dinv, b, g, be, batch2)


def _tc_pool(h3p, co, wo, bo):
    def body(h3_ref, co_ref, wo_ref, bo_ref, o_ref, mean_s, max_s):
        def seg(gi, carry):
            cnt = co_ref[0, gi]
            off = co_ref[1, gi]
            nblk = (cnt + (BK - 1)) // BK

            def blk(i, sm):
                s, mx = sm
                rows = h3_ref[pl.ds(off + i * BK, BK), :]
                rid = lax.broadcasted_iota(jnp.int32, (BK, 1), 0) + i * BK
                valid = rid < cnt
                s = s + jnp.sum(jnp.where(valid, rows, 0.0), axis=0,
                                keepdims=True)
                mx = jnp.maximum(mx, jnp.max(
                    jnp.where(valid, rows, -jnp.inf), axis=0, keepdims=True))
                return s, mx

            s0 = jnp.zeros((1, H), jnp.float32)
            m0 = jnp.full((1, H), -jnp.inf, jnp.float32)
            s, mx = lax.fori_loop(0, nblk, blk, (s0, m0))
            cntf = jnp.maximum(cnt, 1).astype(jnp.float32)
            mean_s[pl.ds(gi, 1), :] = s / cntf
            max_s[pl.ds(gi, 1), :] = jnp.where(cnt > 0, mx, 0.0)
            return carry

        lax.fori_loop(0, G, seg, 0)
        o_ref[...] = (
            jnp.dot(mean_s[...], wo_ref[0:H, :],
                    preferred_element_type=jnp.float32)
            + jnp.dot(max_s[...], wo_ref[H:2 * H, :],
                      preferred_element_type=jnp.float32)
            + bo_ref[...])

    return pl.pallas_call(
        body,
        out_shape=jax.ShapeDtypeStruct((G, 1), jnp.float32),
        in_specs=[
            pl.BlockSpec(memory_space=pltpu.VMEM),
            pl.BlockSpec(memory_space=pltpu.SMEM),
            pl.BlockSpec(memory_space=pltpu.VMEM),
            pl.BlockSpec(memory_space=pltpu.VMEM),
        ],
        scratch_shapes=[
            pltpu.VMEM((G, H), jnp.float32),
            pltpu.VMEM((G, H), jnp.float32),
        ])(h3p, co, wo, bo)


def kernel(x, edge_index, batch, W1, b1, W2, b2, W3, b3,
           g1, be1, g2, be2, g3, be3, Wo, bo):
    srcr = edge_index[0].reshape(NW, NCHUNK, CH)
    dstr = edge_index[1].reshape(NW, NCHUNK, CH)
    batch2 = batch.reshape(N, 1)
    b1r, b2r, b3r = (b.reshape(1, H) for b in (b1, b2, b3))
    g1r, g2r, g3r = (g.reshape(1, H) for g in (g1, g2, g3))
    be1r, be2r, be3r = (b.reshape(1, H) for b in (be1, be2, be3))
    bor = bo.reshape(1, 1)
    zH = jnp.zeros((RPT, H), jnp.float32)
    zD = jnp.zeros((RPT, DW), jnp.float32)
    onesD = jnp.ones((CH, DW), jnp.float32)

    degp = _sc_deg(dstr, onesD, zD)
    z1 = _tc_mm(x, W1)
    dinv, hs1 = _tc_dinv_scale(z1, degp)

    a1 = _sc_agg(hs1, srcr, dstr, zH)
    hs2 = _tc_layer(a1, hs1, dinv, b1r, g1r, be1r, W2)
    a2 = _sc_agg(hs2, srcr, dstr, zH)
    hs3 = _tc_layer(a2, hs2, dinv, b2r, g2r, be2r, W3)
    a3 = _sc_agg(hs3, srcr, dstr, zH)
    h3p, co = _tc_last_layer(a3, hs3, dinv, b3r, g3r, be3r, batch2)

    return _tc_pool(h3p, co, Wo, bor)


# speedup vs baseline: 20.1516x; 20.1516x over previous
import functools

import jax
import jax.numpy as jnp
from jax import lax
from jax.experimental import pallas as pl
from jax.experimental.pallas import tpu as pltpu
from jax.experimental.pallas import tpu_sc as plsc

N = 10000
E = 320000
F_IN = 128
H = 64
G = 64

NC, NS = 2, 16
NW = NC * NS
EPW = E // NW
CH = 80
NCHUNK = EPW // CH
NACC = 10240
RPT = NACC // NS
DW = 16

BK = 240
NPAD = N + BK

_SC_MESH = plsc.VectorSubcoreMesh(core_axis_name="c", subcore_axis_name="s")


def _sc_agg(hs, srcr, dstr, zrows):
    @functools.partial(
        pl.kernel,
        out_type=jax.ShapeDtypeStruct((NC, NACC, H), jnp.float32),
        mesh=_SC_MESH,
        compiler_params=pltpu.CompilerParams(use_tc_tiling_on_sc=False),
        scratch_types=[
            pltpu.VMEM((NCHUNK, CH), jnp.int32),
            pltpu.VMEM((NCHUNK, CH), jnp.int32),
            pltpu.VMEM((CH, H), jnp.float32),
            pltpu.VMEM_SHARED((NACC, H), jnp.float32),
        ],
    )
    def k(hs_hbm, src_hbm, dst_hbm, z_hbm, out_hbm, sslab, dslab, rows, acc):
        cid = lax.axis_index("c")
        sid = lax.axis_index("s")
        wid = sid * NC + cid
        r0 = sid * RPT

        pltpu.sync_copy(z_hbm, acc.at[pl.ds(r0, RPT)])
        pltpu.sync_copy(src_hbm.at[wid], sslab)
        pltpu.sync_copy(dst_hbm.at[wid], dslab)
        plsc.subcore_barrier()

        @pl.loop(0, NCHUNK)
        def _(j):
            pltpu.sync_copy(hs_hbm.at[sslab.at[j]], rows)
            pltpu.sync_copy(rows, acc.at[dslab.at[j]], add=True)

        plsc.subcore_barrier()
        pltpu.sync_copy(acc.at[pl.ds(r0, RPT)], out_hbm.at[cid, pl.ds(r0, RPT)])

    return k(hs, srcr, dstr, zrows)


def _sc_deg(dstr, ones, zrows):
    @functools.partial(
        pl.kernel,
        out_type=jax.ShapeDtypeStruct((NC, NACC, DW), jnp.float32),
        mesh=_SC_MESH,
        compiler_params=pltpu.CompilerParams(use_tc_tiling_on_sc=False),
        scratch_types=[
            pltpu.VMEM((NCHUNK, CH), jnp.int32),
            pltpu.VMEM((CH, DW), jnp.float32),
            pltpu.VMEM_SHARED((NACC, DW), jnp.float32),
        ],
    )
    def k(dst_hbm, ones_hbm, z_hbm, out_hbm, dslab, ones_v, acc):
        cid = lax.axis_index("c")
        sid = lax.axis_index("s")
        wid = sid * NC + cid
        r0 = sid * RPT

        pltpu.sync_copy(z_hbm, acc.at[pl.ds(r0, RPT)])
        pltpu.sync_copy(ones_hbm, ones_v)
        pltpu.sync_copy(dst_hbm.at[wid], dslab)
        plsc.subcore_barrier()

        @pl.loop(0, NCHUNK)
        def _(j):
            pltpu.sync_copy(ones_v, acc.at[dslab.at[j]], add=True)

        plsc.subcore_barrier()
        pltpu.sync_copy(acc.at[pl.ds(r0, RPT)], out_hbm.at[cid, pl.ds(r0, RPT)])

    return k(dstr, ones, zrows)


def _tc_mm(x, w):
    def body(x_ref, w_ref, o_ref):
        o_ref[...] = jnp.dot(x_ref[...], w_ref[...],
                             preferred_element_type=jnp.float32)

    return pl.pallas_call(
        body, out_shape=jax.ShapeDtypeStruct((N, H), jnp.float32))(x, w)


def _tc_dinv_scale(z1, degp):
    def body(z_ref, d_ref, dinv_ref, hs_ref):
        deg = 1.0 + d_ref[0, 0:N, 0:1] + d_ref[1, 0:N, 0:1]
        dinv = lax.rsqrt(deg)
        dinv_ref[...] = dinv
        hs_ref[...] = z_ref[...] * dinv

    return pl.pallas_call(
        body,
        out_shape=(jax.ShapeDtypeStruct((N, 1), jnp.float32),
                   jax.ShapeDtypeStruct((N, H), jnp.float32)))(z1, degp)


def _tc_layer(ap, hs, dinv, b, g, be, wn):
    def body(p_ref, hs_ref, dinv_ref, b_ref, g_ref, be_ref, w_ref, o_ref):
        dv = dinv_ref[...]
        agg = dv * (p_ref[0, 0:N, :] + p_ref[1, 0:N, :] + hs_ref[...]) + b_ref[...]
        m = jnp.mean(agg, axis=0, keepdims=True)
        c = agg - m
        v = jnp.mean(c * c, axis=0, keepdims=True)
        hn = jnp.maximum(c * lax.rsqrt(v + 1e-5) * g_ref[...] + be_ref[...],
                         0.0)
        z = jnp.dot(hn, w_ref[...], preferred_element_type=jnp.float32)
        o_ref[...] = z * dv

    return pl.pallas_call(
        body, out_shape=jax.ShapeDtypeStruct((N, H), jnp.float32))(
            ap, hs, dinv, b, g, be, wn)


def _tc_last_layer(ap, hs, dinv, b, g, be, batch2):
    def body(p_ref, hs_ref, dinv_ref, b_ref, g_ref, be_ref, bt_ref,
             h3_ref, co_ref):
        agg = (dinv_ref[...] * (p_ref[0, 0:N, :] + p_ref[1, 0:N, :] + hs_ref[...])
               + b_ref[...])
        m = jnp.mean(agg, axis=0, keepdims=True)
        c = agg - m
        v = jnp.mean(c * c, axis=0, keepdims=True)
        h3 = jnp.maximum(c * lax.rsqrt(v + 1e-5) * g_ref[...] + be_ref[...],
                         0.0)
        h3_ref[0:N, :] = h3
        h3_ref[N:NPAD, :] = jnp.zeros((NPAD - N, H), jnp.float32)
        bt = bt_ref[...]
        gi = lax.broadcasted_iota(jnp.int32, (N, G), 1)
        co_ref[0:1, :] = jnp.sum((bt == gi).astype(jnp.int32), axis=0,
                                 keepdims=True)
        co_ref[1:2, :] = jnp.sum((bt < gi).astype(jnp.int32), axis=0,
                                 keepdims=True)

    return pl.pallas_call(
        body,
        out_shape=(jax.ShapeDtypeStruct((NPAD, H), jnp.float32),
                   jax.ShapeDtypeStruct((2, G), jnp.int32)))(
            ap, hs, dinv, b, g, be, batch2)


def _tc_pool(h3p, co, wo, bo):
    def body(h3_ref, co_ref, wo_ref, bo_ref, o_ref, mean_s, max_s):
        def seg(gi, carry):
            cnt = co_ref[0, gi]
            off = co_ref[1, gi]
            nblk = (cnt + (BK - 1)) // BK

            def blk(i, sm):
                s, mx = sm
                rows = h3_ref[pl.ds(off + i * BK, BK), :]
                rid = lax.broadcasted_iota(jnp.int32, (BK, 1), 0) + i * BK
                valid = rid < cnt
                s = s + jnp.sum(jnp.where(valid, rows, 0.0), axis=0,
                                keepdims=True)
                mx = jnp.maximum(mx, jnp.max(
                    jnp.where(valid, rows, -jnp.inf), axis=0, keepdims=True))
                return s, mx

            s0 = jnp.zeros((1, H), jnp.float32)
            m0 = jnp.full((1, H), -jnp.inf, jnp.float32)
            s, mx = lax.fori_loop(0, nblk, blk, (s0, m0))
            cntf = jnp.maximum(cnt, 1).astype(jnp.float32)
            mean_s[pl.ds(gi, 1), :] = s / cntf
            max_s[pl.ds(gi, 1), :] = jnp.where(cnt > 0, mx, 0.0)
            return carry

        lax.fori_loop(0, G, seg, 0)
        o_ref[...] = (
            jnp.dot(mean_s[...], wo_ref[0:H, :],
                    preferred_element_type=jnp.float32)
            + jnp.dot(max_s[...], wo_ref[H:2 * H, :],
                      preferred_element_type=jnp.float32)
            + bo_ref[...])

    return pl.pallas_call(
        body,
        out_shape=jax.ShapeDtypeStruct((G, 1), jnp.float32),
        in_specs=[
            pl.BlockSpec(memory_space=pltpu.VMEM),
            pl.BlockSpec(memory_space=pltpu.SMEM),
            pl.BlockSpec(memory_space=pltpu.VMEM),
            pl.BlockSpec(memory_space=pltpu.VMEM),
        ],
        scratch_shapes=[
            pltpu.VMEM((G, H), jnp.float32),
            pltpu.VMEM((G, H), jnp.float32),
        ])(h3p, co, wo, bo)


def kernel(x, edge_index, batch, W1, b1, W2, b2, W3, b3,
           g1, be1, g2, be2, g3, be3, Wo, bo):
    srcr = edge_index[0].reshape(NW, NCHUNK, CH)
    dstr = edge_index[1].reshape(NW, NCHUNK, CH)
    batch2 = batch.reshape(N, 1)
    b1r, b2r, b3r = (b.reshape(1, H) for b in (b1, b2, b3))
    g1r, g2r, g3r = (g.reshape(1, H) for g in (g1, g2, g3))
    be1r, be2r, be3r = (b.reshape(1, H) for b in (be1, be2, be3))
    bor = bo.reshape(1, 1)
    zH = jnp.zeros((RPT, H), jnp.float32)
    zD = jnp.zeros((RPT, DW), jnp.float32)
    onesD = jnp.ones((CH, DW), jnp.float32)

    degp = _sc_deg(dstr, onesD, zD)
    z1 = _tc_mm(x, W1)
    dinv, hs1 = _tc_dinv_scale(z1, degp)

    a1 = _sc_agg(hs1, srcr, dstr, zH)
    hs2 = _tc_layer(a1, hs1, dinv, b1r, g1r, be1r, W2)
    a2 = _sc_agg(hs2, srcr, dstr, zH)
    hs3 = _tc_layer(a2, hs2, dinv, b2r, g2r, be2r, W3)
    a3 = _sc_agg(hs3, srcr, dstr, zH)
    h3p, co = _tc_last_layer(a3, hs3, dinv, b3r, g3r, be3r, batch2)

    return _tc_pool(h3p, co, Wo, bor)


# speedup vs baseline: 32.5221x; 1.6139x over previous
import functools

import jax
import jax.numpy as jnp
from jax import lax
from jax.experimental import pallas as pl
from jax.experimental.pallas import tpu as pltpu
from jax.experimental.pallas import tpu_sc as plsc

N = 10000
E = 320000
F_IN = 128
H = 64
G = 64

NC, NS = 2, 16
NW = NC * NS
CH = 128
NCHUNK = 80
EPW = NCHUNK * CH
E_PAD = NW * EPW
NACC = 10240
RPT = NACC // NS
DW = 16

BK = 240
NPAD = N + BK

_SC_MESH = plsc.VectorSubcoreMesh(core_axis_name="c", subcore_axis_name="s")


def _sc_agg(hs, srcr, dstr, zrows):
    @functools.partial(
        pl.kernel,
        out_type=jax.ShapeDtypeStruct((NC, NACC, H), jnp.float32),
        mesh=_SC_MESH,
        compiler_params=pltpu.CompilerParams(use_tc_tiling_on_sc=False),
        scratch_types=[
            pltpu.VMEM((NCHUNK, CH), jnp.int32),
            pltpu.VMEM((NCHUNK, CH), jnp.int32),
            pltpu.VMEM((CH, H), jnp.float32),
            pltpu.VMEM((CH, H), jnp.float32),
            pltpu.VMEM_SHARED((NACC, H), jnp.float32),
            pltpu.SemaphoreType.DMA,
            pltpu.SemaphoreType.DMA,
        ],
    )
    def k(hs_hbm, src_hbm, dst_hbm, z_hbm, out_hbm,
          sslab, dslab, rows0, rows1, acc, gs0, gs1):
        cid = lax.axis_index("c")
        sid = lax.axis_index("s")
        wid = sid * NC + cid
        r0 = sid * RPT

        pltpu.sync_copy(z_hbm, acc.at[pl.ds(r0, RPT)])
        pltpu.sync_copy(src_hbm.at[wid], sslab)
        pltpu.sync_copy(dst_hbm.at[wid], dslab)
        plsc.subcore_barrier()

        pltpu.async_copy(hs_hbm.at[sslab.at[0]], rows0, gs0)
        pltpu.async_copy(hs_hbm.at[sslab.at[1]], rows1, gs1)

        @pl.loop(0, NCHUNK, step=2)
        def _(j):
            pltpu.make_async_copy(hs_hbm.at[sslab.at[j]], rows0, gs0).wait()
            pltpu.sync_copy(rows0, acc.at[dslab.at[j]], add=True)

            @pl.when(j + 2 < NCHUNK)
            def _():
                pltpu.async_copy(hs_hbm.at[sslab.at[j + 2]], rows0, gs0)

            pltpu.make_async_copy(hs_hbm.at[sslab.at[j + 1]], rows1, gs1).wait()
            pltpu.sync_copy(rows1, acc.at[dslab.at[j + 1]], add=True)

            @pl.when(j + 3 < NCHUNK)
            def _():
                pltpu.async_copy(hs_hbm.at[sslab.at[j + 3]], rows1, gs1)

        plsc.subcore_barrier()
        pltpu.sync_copy(acc.at[pl.ds(r0, RPT)], out_hbm.at[cid, pl.ds(r0, RPT)])

    return k(hs, srcr, dstr, zrows)


def _sc_deg(dstr, ones, zrows):
    @functools.partial(
        pl.kernel,
        out_type=jax.ShapeDtypeStruct((NC, NACC, DW), jnp.float32),
        mesh=_SC_MESH,
        compiler_params=pltpu.CompilerParams(use_tc_tiling_on_sc=False),
        scratch_types=[
            pltpu.VMEM((NCHUNK, CH), jnp.int32),
            pltpu.VMEM((CH, DW), jnp.float32),
            pltpu.VMEM_SHARED((NACC, DW), jnp.float32),
        ],
    )
    def k(dst_hbm, ones_hbm, z_hbm, out_hbm, dslab, ones_v, acc):
        cid = lax.axis_index("c")
        sid = lax.axis_index("s")
        wid = sid * NC + cid
        r0 = sid * RPT

        pltpu.sync_copy(z_hbm, acc.at[pl.ds(r0, RPT)])
        pltpu.sync_copy(ones_hbm, ones_v)
        pltpu.sync_copy(dst_hbm.at[wid], dslab)
        plsc.subcore_barrier()

        @pl.loop(0, NCHUNK)
        def _(j):
            pltpu.sync_copy(ones_v, acc.at[dslab.at[j]], add=True)

        plsc.subcore_barrier()
        pltpu.sync_copy(acc.at[pl.ds(r0, RPT)], out_hbm.at[cid, pl.ds(r0, RPT)])

    return k(dstr, ones, zrows)


def _tc_mm(x, w):
    def body(x_ref, w_ref, o_ref):
        o_ref[...] = jnp.dot(x_ref[...], w_ref[...],
                             preferred_element_type=jnp.float32)

    return pl.pallas_call(
        body, out_shape=jax.ShapeDtypeStruct((N, H), jnp.float32))(x, w)


def _tc_dinv_scale(z1, degp):
    def body(z_ref, d_ref, dinv_ref, hs_ref):
        deg = 1.0 + d_ref[0, 0:N, 0:1] + d_ref[1, 0:N, 0:1]
        dinv = lax.rsqrt(deg)
        dinv_ref[...] = dinv
        hs_ref[...] = z_ref[...] * dinv

    return pl.pallas_call(
        body,
        out_shape=(jax.ShapeDtypeStruct((N, 1), jnp.float32),
                   jax.ShapeDtypeStruct((N, H), jnp.float32)))(z1, degp)


def _tc_layer(ap, hs, dinv, b, g, be, wn):
    def body(p_ref, hs_ref, dinv_ref, b_ref, g_ref, be_ref, w_ref, o_ref):
        dv = dinv_ref[...]
        agg = dv * (p_ref[0, 0:N, :] + p_ref[1, 0:N, :] + hs_ref[...]) + b_ref[...]
        m = jnp.mean(agg, axis=0, keepdims=True)
        c = agg - m
        v = jnp.mean(c * c, axis=0, keepdims=True)
        hn = jnp.maximum(c * lax.rsqrt(v + 1e-5) * g_ref[...] + be_ref[...],
                         0.0)
        z = jnp.dot(hn, w_ref[...], preferred_element_type=jnp.float32)
        o_ref[...] = z * dv

    return pl.pallas_call(
        body, out_shape=jax.ShapeDtypeStruct((N, H), jnp.float32))(
            ap, hs, dinv, b, g, be, wn)


def _tc_last_layer(ap, hs, dinv, b, g, be, batch2):
    def body(p_ref, hs_ref, dinv_ref, b_ref, g_ref, be_ref, bt_ref,
             h3_ref, co_ref):
        agg = (dinv_ref[...] * (p_ref[0, 0:N, :] + p_ref[1, 0:N, :] + hs_ref[...])
               + b_ref[...])
        m = jnp.mean(agg, axis=0, keepdims=True)
        c = agg - m
        v = jnp.mean(c * c, axis=0, keepdims=True)
        h3 = jnp.maximum(c * lax.rsqrt(v + 1e-5) * g_ref[...] + be_ref[...],
                         0.0)
        h3_ref[0:N, :] = h3
        h3_ref[N:NPAD, :] = jnp.zeros((NPAD - N, H), jnp.float32)
        bt = bt_ref[...]
        gi = lax.broadcasted_iota(jnp.int32, (N, G), 1)
        co_ref[0:1, :] = jnp.sum((bt == gi).astype(jnp.int32), axis=0,
                                 keepdims=True)
        co_ref[1:2, :] = jnp.sum((bt < gi).astype(jnp.int32), axis=0,
                                 keepdims=True)

    return pl.pallas_call(
        body,
        out_shape=(jax.ShapeDtypeStruct((NPAD, H), jnp.float32),
                   jax.ShapeDtypeStruct((2, G), jnp.int32)))(
            ap, hs, dinv, b, g, be, batch2)


def _tc_pool(h3p, co, wo, bo):
    def body(h3_ref, co_ref, wo_ref, bo_ref, o_ref, mean_s, max_s):
        def seg(gi, carry):
            cnt = co_ref[0, gi]
            off = co_ref[1, gi]
            nblk = (cnt + (BK - 1)) // BK

            def blk(i, sm):
                s, mx = sm
                rows = h3_ref[pl.ds(off + i * BK, BK), :]
                rid = lax.broadcasted_iota(jnp.int32, (BK, 1), 0) + i * BK
                valid = rid < cnt
                s = s + jnp.sum(jnp.where(valid, rows, 0.0), axis=0,
                                keepdims=True)
                mx = jnp.maximum(mx, jnp.max(
                    jnp.where(valid, rows, -jnp.inf), axis=0, keepdims=True))
                return s, mx

            s0 = jnp.zeros((1, H), jnp.float32)
            m0 = jnp.full((1, H), -jnp.inf, jnp.float32)
            s, mx = lax.fori_loop(0, nblk, blk, (s0, m0))
            cntf = jnp.maximum(cnt, 1).astype(jnp.float32)
            mean_s[pl.ds(gi, 1), :] = s / cntf
            max_s[pl.ds(gi, 1), :] = jnp.where(cnt > 0, mx, 0.0)
            return carry

        lax.fori_loop(0, G, seg, 0)
        o_ref[...] = (
            jnp.dot(mean_s[...], wo_ref[0:H, :],
                    preferred_element_type=jnp.float32)
            + jnp.dot(max_s[...], wo_ref[H:2 * H, :],
                      preferred_element_type=jnp.float32)
            + bo_ref[...])

    return pl.pallas_call(
        body,
        out_shape=jax.ShapeDtypeStruct((G, 1), jnp.float32),
        in_specs=[
            pl.BlockSpec(memory_space=pltpu.VMEM),
            pl.BlockSpec(memory_space=pltpu.SMEM),
            pl.BlockSpec(memory_space=pltpu.VMEM),
            pl.BlockSpec(memory_space=pltpu.VMEM),
        ],
        scratch_shapes=[
            pltpu.VMEM((G, H), jnp.float32),
            pltpu.VMEM((G, H), jnp.float32),
        ])(h3p, co, wo, bo)


def kernel(x, edge_index, batch, W1, b1, W2, b2, W3, b3,
           g1, be1, g2, be2, g3, be3, Wo, bo):
    npad_e = E_PAD - E
    pad_i = jnp.arange(npad_e, dtype=jnp.int32)
    srcr = jnp.concatenate([edge_index[0], pad_i % N]).reshape(NW, NCHUNK, CH)
    dstr = jnp.concatenate([edge_index[1], N + pad_i % (NACC - N)]
                           ).reshape(NW, NCHUNK, CH)
    batch2 = batch.reshape(N, 1)
    b1r, b2r, b3r = (b.reshape(1, H) for b in (b1, b2, b3))
    g1r, g2r, g3r = (g.reshape(1, H) for g in (g1, g2, g3))
    be1r, be2r, be3r = (b.reshape(1, H) for b in (be1, be2, be3))
    bor = bo.reshape(1, 1)
    zH = jnp.zeros((RPT, H), jnp.float32)
    zD = jnp.zeros((RPT, DW), jnp.float32)
    onesD = jnp.ones((CH, DW), jnp.float32)

    degp = _sc_deg(dstr, onesD, zD)
    z1 = _tc_mm(x, W1)
    dinv, hs1 = _tc_dinv_scale(z1, degp)

    a1 = _sc_agg(hs1, srcr, dstr, zH)
    hs2 = _tc_layer(a1, hs1, dinv, b1r, g1r, be1r, W2)
    a2 = _sc_agg(hs2, srcr, dstr, zH)
    hs3 = _tc_layer(a2, hs2, dinv, b2r, g2r, be2r, W3)
    a3 = _sc_agg(hs3, srcr, dstr, zH)
    h3p, co = _tc_last_layer(a3, hs3, dinv, b3r, g3r, be3r, batch2)

    return _tc_pool(h3p, co, Wo, bor)
